# baseline (speedup 1.0000x reference)
import functools

import jax
import jax.numpy as jnp
from jax import lax
from jax.experimental import pallas as pl
from jax.experimental.pallas import tpu as pltpu
from jax.experimental.pallas import tpu_sc as plsc

_CHUNK = 128


@functools.lru_cache(maxsize=None)
def _make_gather(V, D, B):
    info = plsc.get_sparse_core_info()
    NC, NS = info.num_cores, info.num_subcores
    NW = NC * NS
    assert B % (NW * _CHUNK) == 0
    chunks_per_w = B // (NW * _CHUNK)

    mesh = plsc.VectorSubcoreMesh(core_axis_name="c", subcore_axis_name="s")

    @functools.partial(
        pl.kernel,
        mesh=mesh,
        out_type=jax.ShapeDtypeStruct((B, D), jnp.float32),
        scratch_types=[
            pltpu.VMEM((chunks_per_w, _CHUNK), jnp.int32),
            pltpu.VMEM((_CHUNK, D), jnp.float32),
            pltpu.SemaphoreType.DMA,
        ],
        compiler_params=pltpu.CompilerParams(use_tc_tiling_on_sc=False),
    )
    def gather(table_hbm, idx_hbm, out_hbm, idx_v, rows_v, sem):
        wid = lax.axis_index("s") * NC + lax.axis_index("c")
        row_base = wid * chunks_per_w * _CHUNK
        pltpu.sync_copy(idx_hbm.at[pl.ds(wid * chunks_per_w, chunks_per_w)], idx_v)

        def body(j, carry):
            pltpu.async_copy(table_hbm.at[idx_v.at[j]], rows_v, sem).wait()
            pltpu.sync_copy(
                rows_v, out_hbm.at[pl.ds(row_base + j * _CHUNK, _CHUNK)]
            )
            return carry

        lax.fori_loop(0, chunks_per_w, body, 0)

    return gather


def kernel(x, W):
    B = x.shape[0] * x.shape[1]
    V, D = W.shape
    idx = x.reshape(B // _CHUNK, _CHUNK)
    out = _make_gather(V, D, B)(W, idx)
    return out.reshape(x.shape[0], x.shape[1], D)


# speedup vs baseline: 1.1147x; 1.1147x over previous
import functools

import jax
import jax.numpy as jnp
from jax import lax
from jax.experimental import pallas as pl
from jax.experimental.pallas import tpu as pltpu
from jax.experimental.pallas import tpu_sc as plsc

_CHUNK = 128
_K = 4


@functools.lru_cache(maxsize=None)
def _make_gather(V, D, B):
    info = plsc.get_sparse_core_info()
    NC, NS = info.num_cores, info.num_subcores
    NW = NC * NS
    assert B % (NW * _CHUNK) == 0
    chunks_per_w = B // (NW * _CHUNK)
    assert chunks_per_w % (2 * _K) == 0
    pairs = chunks_per_w // (2 * _K)

    mesh = plsc.VectorSubcoreMesh(core_axis_name="c", subcore_axis_name="s")

    @functools.partial(
        pl.kernel,
        mesh=mesh,
        out_type=jax.ShapeDtypeStruct((B, D), jnp.float32),
        scratch_types=[
            pltpu.VMEM((chunks_per_w, _CHUNK), jnp.int32),
            pltpu.VMEM((2, _K, _CHUNK, D), jnp.float32),
            pltpu.SemaphoreType.DMA,
            pltpu.SemaphoreType.DMA,
            pltpu.SemaphoreType.DMA,
            pltpu.SemaphoreType.DMA,
        ],
        compiler_params=pltpu.CompilerParams(use_tc_tiling_on_sc=False),
    )
    def gather(table_hbm, idx_hbm, out_hbm, idx_v, rows_v, g0, g1, w0, w1):
        wid = lax.axis_index("s") * NC + lax.axis_index("c")
        row_base = wid * chunks_per_w * _CHUNK
        pltpu.sync_copy(idx_hbm.at[pl.ds(wid * chunks_per_w, chunks_per_w)], idx_v)

        def fire_g(g, h, sem):
            for b in range(_K):
                pltpu.async_copy(
                    table_hbm.at[idx_v.at[g * _K + b]], rows_v.at[h, b], sem
                )

        def drain_g(h, sem):
            for b in range(_K):
                pltpu.make_async_copy(
                    table_hbm.at[pl.ds(0, _CHUNK)], rows_v.at[h, b], sem
                ).wait()

        def fire_w(g, h, sem):
            for b in range(_K):
                pltpu.async_copy(
                    rows_v.at[h, b],
                    out_hbm.at[pl.ds(row_base + (g * _K + b) * _CHUNK, _CHUNK)],
                    sem,
                )

        def drain_w(h, sem):
            for b in range(_K):
                pltpu.make_async_copy(
                    rows_v.at[h, b], out_hbm.at[pl.ds(row_base, _CHUNK)], sem
                ).wait()

        fire_g(0, 0, g0)

        def body(t, carry):
            @pl.when(t > 0)
            def _():
                drain_w(1, w1)

            fire_g(2 * t + 1, 1, g1)
            drain_g(0, g0)
            fire_w(2 * t, 0, w0)
            drain_w(0, w0)

            @pl.when(t < pairs - 1)
            def _():
                fire_g(2 * t + 2, 0, g0)

            drain_g(1, g1)
            fire_w(2 * t + 1, 1, w1)
            return carry

        lax.fori_loop(0, pairs, body, 0)
        drain_w(1, w1)

    return gather


def kernel(x, W):
    B = x.shape[0] * x.shape[1]
    V, D = W.shape
    idx = x.reshape(B // _CHUNK, _CHUNK)
    out = _make_gather(V, D, B)(W, idx)
    return out.reshape(x.shape[0], x.shape[1], D)
